# baseline (speedup 1.0000x reference)
import functools

import jax
import jax.numpy as jnp
from jax import lax
from jax.experimental import pallas as pl
from jax.experimental.pallas import tpu as pltpu

_H = 64
_M = _H * _H
_OC = 64
_IC = 128
_K1 = 9 * (_IC + _OC)
_K2 = 9 * _OC
_EPS = 1e-5


def _edge_map(w_one_rows, base_row):
    top = w_one_rows[0] + w_one_rows[1] + w_one_rows[2]
    bot = w_one_rows[6] + w_one_rows[7] + w_one_rows[8]
    lef = w_one_rows[0] + w_one_rows[3] + w_one_rows[6]
    rig = w_one_rows[2] + w_one_rows[5] + w_one_rows[8]
    full = base_row + sum(w_one_rows[1:], w_one_rows[0])
    hh = lax.broadcasted_iota(jnp.int32, (_M, 1), 0) // _H
    ww = lax.broadcasted_iota(jnp.int32, (_M, 1), 0) % _H
    h0 = (hh == 0).astype(jnp.float32)
    h1 = (hh == _H - 1).astype(jnp.float32)
    w0 = (ww == 0).astype(jnp.float32)
    w1 = (ww == _H - 1).astype(jnp.float32)
    e = jnp.concatenate(
        [h0, h1, w0, w1, h0 * w0, h0 * w1, h1 * w0, h1 * w1], axis=1)
    corr = jnp.concatenate(
        [-top, -bot, -lef, -rig,
         w_one_rows[0], w_one_rows[2], w_one_rows[6], w_one_rows[8]], axis=0)
    return (jnp.broadcast_to(full, (_M, _OC))
            + jnp.dot(e, corr, preferred_element_type=jnp.float32))


def _bn_rows(st, g_row, b_row, count):
    mean = st[0:1] / count
    var = jnp.maximum(st[1:2] / count - mean * mean, 0.0)
    scale = g_row * lax.rsqrt(var + _EPS)
    shift = b_row - mean * scale
    return scale, shift


def _prep1_body(up_w_ref, up_b_ref, c1_w_ref, c1_b_ref, w_ref, bmap_ref):
    w_ones = []
    for t in range(9):
        wu = c1_w_ref[t, :_OC, :]
        w_ref[t * _IC:(t + 1) * _IC, :] = jnp.dot(
            up_w_ref[...], wu, preferred_element_type=jnp.float32
        ).astype(jnp.bfloat16)
        w_ref[9 * _IC + t * _OC:9 * _IC + (t + 1) * _OC, :] = (
            c1_w_ref[t, _OC:, :].astype(jnp.bfloat16))
        w_ones.append(jnp.dot(up_b_ref[...], wu,
                              preferred_element_type=jnp.float32))
    bmap_ref[...] = _edge_map(w_ones, c1_b_ref[...])


def _prep1_call(up_w, up_b, c1_w, c1_b):
    return pl.pallas_call(
        _prep1_body,
        out_shape=(jax.ShapeDtypeStruct((_K1, _OC), jnp.bfloat16),
                   jax.ShapeDtypeStruct((_M, _OC), jnp.float32)),
        in_specs=[pl.BlockSpec(a.shape, lambda nd=a.ndim: (0,) * nd)
                  for a in (up_w, up_b, c1_w, c1_b)],
        out_specs=(pl.BlockSpec((_K1, _OC), lambda: (0, 0)),
                   pl.BlockSpec((_M, _OC), lambda: (0, 0))),
    )(up_w, up_b, c1_w, c1_b)


def _prep2_body(st_ref, g1_ref, b1_ref, c2_w_ref, c2_b_ref, w_ref, bmap_ref,
                count):
    st = jnp.sum(st_ref[...], axis=0)
    s1, t1 = _bn_rows(st, g1_ref[...], b1_ref[...], count)
    s1c = jnp.transpose(s1)
    w_ones = []
    for t in range(9):
        wt = c2_w_ref[t]
        w_ref[t * _OC:(t + 1) * _OC, :] = (wt * s1c).astype(jnp.bfloat16)
        w_ones.append(jnp.dot(t1, wt, preferred_element_type=jnp.float32))
    bmap_ref[...] = _edge_map(w_ones, c2_b_ref[...])


def _prep2_call(st_all, g1, b1, c2_w, c2_b, count):
    return pl.pallas_call(
        functools.partial(_prep2_body, count=count),
        out_shape=(jax.ShapeDtypeStruct((_K2, _OC), jnp.bfloat16),
                   jax.ShapeDtypeStruct((_M, _OC), jnp.float32)),
        in_specs=[pl.BlockSpec(a.shape, lambda nd=a.ndim: (0,) * nd)
                  for a in (st_all, g1, b1, c2_w, c2_b)],
        out_specs=(pl.BlockSpec((_K2, _OC), lambda: (0, 0)),
                   pl.BlockSpec((_M, _OC), lambda: (0, 0))),
    )(st_all, g1, b1, c2_w, c2_b)


def _upsample2x(xt):
    h = xt.shape[0]
    prev = jnp.concatenate([xt[0:1], xt[:-1]], axis=0)
    nxt = jnp.concatenate([xt[1:], xt[-1:]], axis=0)
    xh = jnp.stack([0.75 * xt + 0.25 * prev, 0.75 * xt + 0.25 * nxt],
                   axis=1).reshape(2 * h, h, xt.shape[2])
    prevw = jnp.concatenate([xh[:, 0:1], xh[:, :-1]], axis=1)
    nxtw = jnp.concatenate([xh[:, 1:], xh[:, -1:]], axis=1)
    return jnp.stack([0.75 * xh + 0.25 * prevw, 0.75 * xh + 0.25 * nxtw],
                     axis=2).reshape(2 * h, 2 * h, xt.shape[2])


def _relu_stats_store(acc, bmap_ref, y_ref, st_ref):
    y = jnp.maximum(acc + bmap_ref[...], 0.0)
    y_ref[...] = y.astype(jnp.bfloat16)
    st_ref[...] = jnp.concatenate(
        [jnp.sum(y, axis=0, keepdims=True),
         jnp.sum(y * y, axis=0, keepdims=True)], axis=0)


_B = 8
_B1 = 2


def _conv1_body(x_ref, sk_ref, w_ref, bmap_ref, y_ref, st_ref, xpx, xps, col):
    xpx[:, 0, :] = jnp.zeros_like(xpx[:, 0, :])
    xpx[:, _H + 1, :] = jnp.zeros_like(xpx[:, _H + 1, :])
    xpx[0, :, :] = jnp.zeros_like(xpx[0, :, :])
    xpx[_H + 1, :, :] = jnp.zeros_like(xpx[_H + 1, :, :])
    xps[:, 0, :] = jnp.zeros_like(xps[:, 0, :])
    xps[:, _H + 1, :] = jnp.zeros_like(xps[:, _H + 1, :])
    xps[0, :, :] = jnp.zeros_like(xps[0, :, :])
    xps[_H + 1, :, :] = jnp.zeros_like(xps[_H + 1, :, :])
    for img in range(_B1):
        xu = _upsample2x(x_ref[img].astype(jnp.float32))
        xpx[1:_H + 1, 1:_H + 1, :] = xu.astype(jnp.bfloat16)
        xps[1:_H + 1, 1:_H + 1, :] = sk_ref[img]
        for t in range(9):
            dy, dx = divmod(t, 3)
            col[:, :, t * _IC:(t + 1) * _IC] = xpx[dy:dy + _H, dx:dx + _H, :]
            col[:, :, 9 * _IC + t * _OC:9 * _IC + (t + 1) * _OC] = (
                xps[dy:dy + _H, dx:dx + _H, :])
        acc = jnp.dot(col[...].reshape(_M, _K1), w_ref[...],
                      preferred_element_type=jnp.float32)
        _relu_stats_store(acc, bmap_ref, y_ref.at[img], st_ref.at[img])


def _conv2_body(y1_ref, w_ref, bmap_ref, y_ref, st_ref, xpy, col):
    xpy[:, 0, :] = jnp.zeros_like(xpy[:, 0, :])
    xpy[:, _H + 1, :] = jnp.zeros_like(xpy[:, _H + 1, :])
    xpy[0, :, :] = jnp.zeros_like(xpy[0, :, :])
    xpy[_H + 1, :, :] = jnp.zeros_like(xpy[_H + 1, :, :])
    for img in range(_B):
        xpy[1:_H + 1, 1:_H + 1, :] = y1_ref[img]
        for t in range(9):
            dy, dx = divmod(t, 3)
            col[:, :, t * _OC:(t + 1) * _OC] = xpy[dy:dy + _H, dx:dx + _H, :]
        acc = jnp.dot(col[...].reshape(_M, _K2), w_ref[...],
                      preferred_element_type=jnp.float32)
        _relu_stats_store(acc, bmap_ref, y_ref.at[img], st_ref.at[img])


def _conv1_call(x, skip, w, bmap):
    n = x.shape[0]
    return pl.pallas_call(
        _conv1_body,
        out_shape=(jax.ShapeDtypeStruct((n, _M, _OC), jnp.bfloat16),
                   jax.ShapeDtypeStruct((n, 2, _OC), jnp.float32)),
        grid=(n // _B1,),
        in_specs=[
            pl.BlockSpec((_B1, _H // 2, _H // 2, _IC), lambda i: (i, 0, 0, 0)),
            pl.BlockSpec((_B1, _H, _H, _OC), lambda i: (i, 0, 0, 0)),
            pl.BlockSpec((_K1, _OC), lambda i: (0, 0)),
            pl.BlockSpec((_M, _OC), lambda i: (0, 0)),
        ],
        out_specs=(
            pl.BlockSpec((_B1, _M, _OC), lambda i: (i, 0, 0)),
            pl.BlockSpec((_B1, 2, _OC), lambda i: (i, 0, 0)),
        ),
        scratch_shapes=[
            pltpu.VMEM((_H + 2, _H + 2, _IC), jnp.bfloat16),
            pltpu.VMEM((_H + 2, _H + 2, _OC), jnp.bfloat16),
            pltpu.VMEM((_H, _H, _K1), jnp.bfloat16),
        ],
        compiler_params=pltpu.CompilerParams(
            dimension_semantics=("parallel",)),
    )(x, skip, w, bmap)


def _conv2_call(y1, w, bmap):
    n = y1.shape[0]
    return pl.pallas_call(
        _conv2_body,
        out_shape=(jax.ShapeDtypeStruct((n, _M, _OC), jnp.bfloat16),
                   jax.ShapeDtypeStruct((n, 2, _OC), jnp.float32)),
        grid=(n // _B,),
        in_specs=[
            pl.BlockSpec((_B, _H, _H, _OC), lambda i: (i, 0, 0, 0)),
            pl.BlockSpec((_K2, _OC), lambda i: (0, 0)),
            pl.BlockSpec((_M, _OC), lambda i: (0, 0)),
        ],
        out_specs=(
            pl.BlockSpec((_B, _M, _OC), lambda i: (i, 0, 0)),
            pl.BlockSpec((_B, 2, _OC), lambda i: (i, 0, 0)),
        ),
        scratch_shapes=[
            pltpu.VMEM((_H + 2, _H + 2, _OC), jnp.bfloat16),
            pltpu.VMEM((_H, _H, _K2), jnp.bfloat16),
        ],
        compiler_params=pltpu.CompilerParams(
            dimension_semantics=("parallel",)),
    )(y1, w, bmap)


def _epi_body(st_ref, g2_ref, b2_ref, y2_ref, o_ref, count):
    st = jnp.sum(st_ref[...], axis=0)
    s2, t2 = _bn_rows(st, g2_ref[...], b2_ref[...], count)
    for img in range(_B):
        z = y2_ref[img].astype(jnp.float32) * s2 + t2
        o_ref[img] = jnp.transpose(z.reshape(_H, _H, _OC), (2, 0, 1))


def _epi_call(st_all, g2, b2, y2, count):
    n = y2.shape[0]
    return pl.pallas_call(
        functools.partial(_epi_body, count=count),
        out_shape=jax.ShapeDtypeStruct((n, _OC, _H, _H), jnp.float32),
        grid=(n // _B,),
        in_specs=[
            pl.BlockSpec((n, 2, _OC), lambda i: (0, 0, 0)),
            pl.BlockSpec((1, _OC), lambda i: (0, 0)),
            pl.BlockSpec((1, _OC), lambda i: (0, 0)),
            pl.BlockSpec((_B, _M, _OC), lambda i: (i, 0, 0)),
        ],
        out_specs=pl.BlockSpec((_B, _OC, _H, _H), lambda i: (i, 0, 0, 0)),
        compiler_params=pltpu.CompilerParams(
            dimension_semantics=("parallel",)),
    )(st_all, g2, b2, y2)


def kernel(up_w, up_b, c1_w, c1_b, c2_w, c2_b,
           bn1_g, bn1_b, bn2_g, bn2_b, x, skip):
    n = x.shape[0]
    count = float(n * _M)

    x_nhwc = jnp.transpose(x, (0, 2, 3, 1)).astype(jnp.bfloat16)
    sk_nhwc = jnp.transpose(skip, (0, 2, 3, 1)).astype(jnp.bfloat16)
    w1, bmap1 = _prep1_call(up_w, up_b[None, :], c1_w, c1_b[None, :])
    y1, st1 = _conv1_call(x_nhwc, sk_nhwc, w1, bmap1)

    w2, bmap2 = _prep2_call(st1, bn1_g[None, :], bn1_b[None, :],
                            c2_w, c2_b[None, :], count)
    y2, st2 = _conv2_call(y1.reshape(n, _H, _H, _OC), w2, bmap2)

    return _epi_call(st2, bn2_g[None, :], bn2_b[None, :], y2, count)


# speedup vs baseline: 1.1380x; 1.1380x over previous
import functools

import jax
import jax.numpy as jnp
from jax import lax
from jax.experimental import pallas as pl
from jax.experimental.pallas import tpu as pltpu

_H = 64
_M = _H * _H
_OC = 64
_IC = 128
_K1 = 9 * (_IC + _OC)
_K2 = 9 * _OC
_EPS = 1e-5


def _edge_map(w_one_rows, base_row):
    top = w_one_rows[0] + w_one_rows[1] + w_one_rows[2]
    bot = w_one_rows[6] + w_one_rows[7] + w_one_rows[8]
    lef = w_one_rows[0] + w_one_rows[3] + w_one_rows[6]
    rig = w_one_rows[2] + w_one_rows[5] + w_one_rows[8]
    full = base_row + sum(w_one_rows[1:], w_one_rows[0])
    hh = lax.broadcasted_iota(jnp.int32, (_M, 1), 0) // _H
    ww = lax.broadcasted_iota(jnp.int32, (_M, 1), 0) % _H
    h0 = (hh == 0).astype(jnp.float32)
    h1 = (hh == _H - 1).astype(jnp.float32)
    w0 = (ww == 0).astype(jnp.float32)
    w1 = (ww == _H - 1).astype(jnp.float32)
    e = jnp.concatenate(
        [h0, h1, w0, w1, h0 * w0, h0 * w1, h1 * w0, h1 * w1], axis=1)
    corr = jnp.concatenate(
        [-top, -bot, -lef, -rig,
         w_one_rows[0], w_one_rows[2], w_one_rows[6], w_one_rows[8]], axis=0)
    return (jnp.broadcast_to(full, (_M, _OC))
            + jnp.dot(e, corr, preferred_element_type=jnp.float32))


def _bn_rows(st, g_row, b_row, count):
    mean = st[0:1] / count
    var = jnp.maximum(st[1:2] / count - mean * mean, 0.0)
    scale = g_row * lax.rsqrt(var + _EPS)
    shift = b_row - mean * scale
    return scale, shift


def _prep1_body(up_w_ref, up_b_ref, c1_w_ref, c1_b_ref, w_ref, bmap_ref):
    w_ones = []
    for t in range(9):
        wu = c1_w_ref[t, :_OC, :]
        w_ref[t * _IC:(t + 1) * _IC, :] = jnp.dot(
            up_w_ref[...], wu, preferred_element_type=jnp.float32
        ).astype(jnp.bfloat16)
        w_ref[9 * _IC + t * _OC:9 * _IC + (t + 1) * _OC, :] = (
            c1_w_ref[t, _OC:, :].astype(jnp.bfloat16))
        w_ones.append(jnp.dot(up_b_ref[...], wu,
                              preferred_element_type=jnp.float32))
    bmap_ref[...] = _edge_map(w_ones, c1_b_ref[...])


def _prep1_call(up_w, up_b, c1_w, c1_b):
    return pl.pallas_call(
        _prep1_body,
        out_shape=(jax.ShapeDtypeStruct((_K1, _OC), jnp.bfloat16),
                   jax.ShapeDtypeStruct((_M, _OC), jnp.float32)),
        in_specs=[pl.BlockSpec(a.shape, lambda nd=a.ndim: (0,) * nd)
                  for a in (up_w, up_b, c1_w, c1_b)],
        out_specs=(pl.BlockSpec((_K1, _OC), lambda: (0, 0)),
                   pl.BlockSpec((_M, _OC), lambda: (0, 0))),
    )(up_w, up_b, c1_w, c1_b)


def _prep2_body(st_ref, g1_ref, b1_ref, c2_w_ref, c2_b_ref, w_ref, bmap_ref,
                count):
    st = jnp.sum(st_ref[...], axis=0)
    s1, t1 = _bn_rows(st, g1_ref[...], b1_ref[...], count)
    s1c = jnp.transpose(s1)
    w_ones = []
    for t in range(9):
        wt = c2_w_ref[t]
        w_ref[t * _OC:(t + 1) * _OC, :] = (wt * s1c).astype(jnp.bfloat16)
        w_ones.append(jnp.dot(t1, wt, preferred_element_type=jnp.float32))
    bmap_ref[...] = _edge_map(w_ones, c2_b_ref[...])


def _prep2_call(st_all, g1, b1, c2_w, c2_b, count):
    return pl.pallas_call(
        functools.partial(_prep2_body, count=count),
        out_shape=(jax.ShapeDtypeStruct((_K2, _OC), jnp.bfloat16),
                   jax.ShapeDtypeStruct((_M, _OC), jnp.float32)),
        in_specs=[pl.BlockSpec(a.shape, lambda nd=a.ndim: (0,) * nd)
                  for a in (st_all, g1, b1, c2_w, c2_b)],
        out_specs=(pl.BlockSpec((_K2, _OC), lambda: (0, 0)),
                   pl.BlockSpec((_M, _OC), lambda: (0, 0))),
    )(st_all, g1, b1, c2_w, c2_b)


def _upsample2x(xt):
    h = xt.shape[0]
    prev = jnp.concatenate([xt[0:1], xt[:-1]], axis=0)
    nxt = jnp.concatenate([xt[1:], xt[-1:]], axis=0)
    xh = jnp.stack([0.75 * xt + 0.25 * prev, 0.75 * xt + 0.25 * nxt],
                   axis=1).reshape(2 * h, h, xt.shape[2])
    prevw = jnp.concatenate([xh[:, 0:1], xh[:, :-1]], axis=1)
    nxtw = jnp.concatenate([xh[:, 1:], xh[:, -1:]], axis=1)
    return jnp.stack([0.75 * xh + 0.25 * prevw, 0.75 * xh + 0.25 * nxtw],
                     axis=2).reshape(2 * h, 2 * h, xt.shape[2])


def _relu_stats_store(acc, bmap_ref, y_ref, st_ref):
    y = jnp.maximum(acc + bmap_ref[...], 0.0)
    y_ref[...] = y.astype(jnp.bfloat16)
    st_ref[...] = jnp.concatenate(
        [jnp.sum(y, axis=0, keepdims=True),
         jnp.sum(y * y, axis=0, keepdims=True)], axis=0)


_B = 2
_B1 = 2


def _conv1_body(x_ref, sk_ref, w_ref, bmap_ref, y_ref, st_ref, xpx, xps, col):
    xpx[:, 0, :] = jnp.zeros_like(xpx[:, 0, :])
    xpx[:, _H + 1, :] = jnp.zeros_like(xpx[:, _H + 1, :])
    xpx[0, :, :] = jnp.zeros_like(xpx[0, :, :])
    xpx[_H + 1, :, :] = jnp.zeros_like(xpx[_H + 1, :, :])
    xps[:, 0, :] = jnp.zeros_like(xps[:, 0, :])
    xps[:, _H + 1, :] = jnp.zeros_like(xps[:, _H + 1, :])
    xps[0, :, :] = jnp.zeros_like(xps[0, :, :])
    xps[_H + 1, :, :] = jnp.zeros_like(xps[_H + 1, :, :])
    for img in range(_B1):
        xu = _upsample2x(x_ref[img].astype(jnp.float32))
        xpx[1:_H + 1, 1:_H + 1, :] = xu.astype(jnp.bfloat16)
        xps[1:_H + 1, 1:_H + 1, :] = sk_ref[img]
        for t in range(9):
            dy, dx = divmod(t, 3)
            col[:, :, t * _IC:(t + 1) * _IC] = xpx[dy:dy + _H, dx:dx + _H, :]
            col[:, :, 9 * _IC + t * _OC:9 * _IC + (t + 1) * _OC] = (
                xps[dy:dy + _H, dx:dx + _H, :])
        acc = jnp.dot(col[...].reshape(_M, _K1), w_ref[...],
                      preferred_element_type=jnp.float32)
        _relu_stats_store(acc, bmap_ref, y_ref.at[img], st_ref.at[img])


def _conv2_body(y1_ref, w_ref, bmap_ref, y_ref, st_ref, xpy, col):
    xpy[:, 0, :] = jnp.zeros_like(xpy[:, 0, :])
    xpy[:, _H + 1, :] = jnp.zeros_like(xpy[:, _H + 1, :])
    xpy[0, :, :] = jnp.zeros_like(xpy[0, :, :])
    xpy[_H + 1, :, :] = jnp.zeros_like(xpy[_H + 1, :, :])
    for img in range(_B):
        xpy[1:_H + 1, 1:_H + 1, :] = y1_ref[img]
        for t in range(9):
            dy, dx = divmod(t, 3)
            col[:, :, t * _OC:(t + 1) * _OC] = xpy[dy:dy + _H, dx:dx + _H, :]
        acc = jnp.dot(col[...].reshape(_M, _K2), w_ref[...],
                      preferred_element_type=jnp.float32)
        _relu_stats_store(acc, bmap_ref, y_ref.at[img], st_ref.at[img])


def _conv1_call(x, skip, w, bmap):
    n = x.shape[0]
    return pl.pallas_call(
        _conv1_body,
        out_shape=(jax.ShapeDtypeStruct((n, _M, _OC), jnp.bfloat16),
                   jax.ShapeDtypeStruct((n, 2, _OC), jnp.float32)),
        grid=(n // _B1,),
        in_specs=[
            pl.BlockSpec((_B1, _H // 2, _H // 2, _IC), lambda i: (i, 0, 0, 0)),
            pl.BlockSpec((_B1, _H, _H, _OC), lambda i: (i, 0, 0, 0)),
            pl.BlockSpec((_K1, _OC), lambda i: (0, 0)),
            pl.BlockSpec((_M, _OC), lambda i: (0, 0)),
        ],
        out_specs=(
            pl.BlockSpec((_B1, _M, _OC), lambda i: (i, 0, 0)),
            pl.BlockSpec((_B1, 2, _OC), lambda i: (i, 0, 0)),
        ),
        scratch_shapes=[
            pltpu.VMEM((_H + 2, _H + 2, _IC), jnp.bfloat16),
            pltpu.VMEM((_H + 2, _H + 2, _OC), jnp.bfloat16),
            pltpu.VMEM((_H, _H, _K1), jnp.bfloat16),
        ],
        compiler_params=pltpu.CompilerParams(
            dimension_semantics=("parallel",)),
    )(x, skip, w, bmap)


def _conv2_call(y1, w, bmap):
    n = y1.shape[0]
    return pl.pallas_call(
        _conv2_body,
        out_shape=(jax.ShapeDtypeStruct((n, _M, _OC), jnp.bfloat16),
                   jax.ShapeDtypeStruct((n, 2, _OC), jnp.float32)),
        grid=(n // _B,),
        in_specs=[
            pl.BlockSpec((_B, _H, _H, _OC), lambda i: (i, 0, 0, 0)),
            pl.BlockSpec((_K2, _OC), lambda i: (0, 0)),
            pl.BlockSpec((_M, _OC), lambda i: (0, 0)),
        ],
        out_specs=(
            pl.BlockSpec((_B, _M, _OC), lambda i: (i, 0, 0)),
            pl.BlockSpec((_B, 2, _OC), lambda i: (i, 0, 0)),
        ),
        scratch_shapes=[
            pltpu.VMEM((_H + 2, _H + 2, _OC), jnp.bfloat16),
            pltpu.VMEM((_H, _H, _K2), jnp.bfloat16),
        ],
        compiler_params=pltpu.CompilerParams(
            dimension_semantics=("parallel",)),
    )(y1, w, bmap)


def _epi_body(st_ref, g2_ref, b2_ref, y2_ref, o_ref, count):
    st = jnp.sum(st_ref[...], axis=0)
    s2, t2 = _bn_rows(st, g2_ref[...], b2_ref[...], count)
    for img in range(_B):
        z = y2_ref[img].astype(jnp.float32) * s2 + t2
        o_ref[img] = jnp.transpose(z.reshape(_H, _H, _OC), (2, 0, 1))


def _epi_call(st_all, g2, b2, y2, count):
    n = y2.shape[0]
    return pl.pallas_call(
        functools.partial(_epi_body, count=count),
        out_shape=jax.ShapeDtypeStruct((n, _OC, _H, _H), jnp.float32),
        grid=(n // _B,),
        in_specs=[
            pl.BlockSpec((n, 2, _OC), lambda i: (0, 0, 0)),
            pl.BlockSpec((1, _OC), lambda i: (0, 0)),
            pl.BlockSpec((1, _OC), lambda i: (0, 0)),
            pl.BlockSpec((_B, _M, _OC), lambda i: (i, 0, 0)),
        ],
        out_specs=pl.BlockSpec((_B, _OC, _H, _H), lambda i: (i, 0, 0, 0)),
        compiler_params=pltpu.CompilerParams(
            dimension_semantics=("parallel",)),
    )(st_all, g2, b2, y2)


def kernel(up_w, up_b, c1_w, c1_b, c2_w, c2_b,
           bn1_g, bn1_b, bn2_g, bn2_b, x, skip):
    n = x.shape[0]
    count = float(n * _M)

    x_nhwc = jnp.transpose(x, (0, 2, 3, 1)).astype(jnp.bfloat16)
    sk_nhwc = jnp.transpose(skip, (0, 2, 3, 1)).astype(jnp.bfloat16)
    w1, bmap1 = _prep1_call(up_w, up_b[None, :], c1_w, c1_b[None, :])
    y1, st1 = _conv1_call(x_nhwc, sk_nhwc, w1, bmap1)

    w2, bmap2 = _prep2_call(st1, bn1_g[None, :], bn1_b[None, :],
                            c2_w, c2_b[None, :], count)
    y2, st2 = _conv2_call(y1.reshape(n, _H, _H, _OC), w2, bmap2)

    return _epi_call(st2, bn2_g[None, :], bn2_b[None, :], y2, count)


# speedup vs baseline: 1.1589x; 1.0184x over previous
import functools

import jax
import jax.numpy as jnp
from jax import lax
from jax.experimental import pallas as pl
from jax.experimental.pallas import tpu as pltpu

_H = 64
_M = _H * _H
_OC = 64
_IC = 128
_K1 = 9 * (_IC + _OC)
_K2 = 9 * _OC
_EPS = 1e-5


def _edge_map(w_one_rows, base_row):
    top = w_one_rows[0] + w_one_rows[1] + w_one_rows[2]
    bot = w_one_rows[6] + w_one_rows[7] + w_one_rows[8]
    lef = w_one_rows[0] + w_one_rows[3] + w_one_rows[6]
    rig = w_one_rows[2] + w_one_rows[5] + w_one_rows[8]
    full = base_row + sum(w_one_rows[1:], w_one_rows[0])
    hh = lax.broadcasted_iota(jnp.int32, (_M, 1), 0) // _H
    ww = lax.broadcasted_iota(jnp.int32, (_M, 1), 0) % _H
    h0 = (hh == 0).astype(jnp.float32)
    h1 = (hh == _H - 1).astype(jnp.float32)
    w0 = (ww == 0).astype(jnp.float32)
    w1 = (ww == _H - 1).astype(jnp.float32)
    e = jnp.concatenate(
        [h0, h1, w0, w1, h0 * w0, h0 * w1, h1 * w0, h1 * w1], axis=1)
    corr = jnp.concatenate(
        [-top, -bot, -lef, -rig,
         w_one_rows[0], w_one_rows[2], w_one_rows[6], w_one_rows[8]], axis=0)
    return (jnp.broadcast_to(full, (_M, _OC))
            + jnp.dot(e, corr, preferred_element_type=jnp.float32))


def _bn_rows(st, g_row, b_row, count):
    mean = st[0:1] / count
    var = jnp.maximum(st[1:2] / count - mean * mean, 0.0)
    scale = g_row * lax.rsqrt(var + _EPS)
    shift = b_row - mean * scale
    return scale, shift


def _prep1_body(up_w_ref, up_b_ref, c1_w_ref, c1_b_ref, w_ref, bmap_ref):
    w_ones = []
    for t in range(9):
        wu = c1_w_ref[t, :_OC, :]
        w_ref[t * _IC:(t + 1) * _IC, :] = jnp.dot(
            up_w_ref[...], wu, preferred_element_type=jnp.float32
        ).astype(jnp.bfloat16)
        w_ref[9 * _IC + t * _OC:9 * _IC + (t + 1) * _OC, :] = (
            c1_w_ref[t, _OC:, :].astype(jnp.bfloat16))
        w_ones.append(jnp.dot(up_b_ref[...], wu,
                              preferred_element_type=jnp.float32))
    bmap_ref[...] = _edge_map(w_ones, c1_b_ref[...])


def _prep1_call(up_w, up_b, c1_w, c1_b):
    return pl.pallas_call(
        _prep1_body,
        out_shape=(jax.ShapeDtypeStruct((_K1, _OC), jnp.bfloat16),
                   jax.ShapeDtypeStruct((_M, _OC), jnp.float32)),
        in_specs=[pl.BlockSpec(a.shape, lambda nd=a.ndim: (0,) * nd)
                  for a in (up_w, up_b, c1_w, c1_b)],
        out_specs=(pl.BlockSpec((_K1, _OC), lambda: (0, 0)),
                   pl.BlockSpec((_M, _OC), lambda: (0, 0))),
    )(up_w, up_b, c1_w, c1_b)


def _prep2_body(st_ref, g1_ref, b1_ref, c2_w_ref, c2_b_ref, w_ref, bmap_ref,
                count):
    st = jnp.sum(st_ref[...], axis=0)
    s1, t1 = _bn_rows(st, g1_ref[...], b1_ref[...], count)
    s1c = jnp.transpose(s1)
    w_ones = []
    for t in range(9):
        wt = c2_w_ref[t]
        w_ref[t * _OC:(t + 1) * _OC, :] = (wt * s1c).astype(jnp.bfloat16)
        w_ones.append(jnp.dot(t1, wt, preferred_element_type=jnp.float32))
    bmap_ref[...] = _edge_map(w_ones, c2_b_ref[...])


def _prep2_call(st_all, g1, b1, c2_w, c2_b, count):
    return pl.pallas_call(
        functools.partial(_prep2_body, count=count),
        out_shape=(jax.ShapeDtypeStruct((_K2, _OC), jnp.bfloat16),
                   jax.ShapeDtypeStruct((_M, _OC), jnp.float32)),
        in_specs=[pl.BlockSpec(a.shape, lambda nd=a.ndim: (0,) * nd)
                  for a in (st_all, g1, b1, c2_w, c2_b)],
        out_specs=(pl.BlockSpec((_K2, _OC), lambda: (0, 0)),
                   pl.BlockSpec((_M, _OC), lambda: (0, 0))),
    )(st_all, g1, b1, c2_w, c2_b)


def _upsample2x(xt):
    h = xt.shape[0]
    prev = jnp.concatenate([xt[0:1], xt[:-1]], axis=0)
    nxt = jnp.concatenate([xt[1:], xt[-1:]], axis=0)
    xh = jnp.stack([0.75 * xt + 0.25 * prev, 0.75 * xt + 0.25 * nxt],
                   axis=1).reshape(2 * h, h, xt.shape[2])
    prevw = jnp.concatenate([xh[:, 0:1], xh[:, :-1]], axis=1)
    nxtw = jnp.concatenate([xh[:, 1:], xh[:, -1:]], axis=1)
    return jnp.stack([0.75 * xh + 0.25 * prevw, 0.75 * xh + 0.25 * nxtw],
                     axis=2).reshape(2 * h, 2 * h, xt.shape[2])


def _relu_stats_store(acc, bmap_ref, y_ref, st_ref):
    y = jnp.maximum(acc + bmap_ref[...], 0.0)
    y_ref[...] = y.astype(jnp.bfloat16)
    st_ref[...] = jnp.concatenate(
        [jnp.sum(y, axis=0, keepdims=True),
         jnp.sum(y * y, axis=0, keepdims=True)], axis=0)


_B = 2
_B1 = 1


def _conv1_body(x_ref, sk_ref, w_ref, bmap_ref, y_ref, st_ref, xpx, xps, col):
    xpx[:, 0, :] = jnp.zeros_like(xpx[:, 0, :])
    xpx[:, _H + 1, :] = jnp.zeros_like(xpx[:, _H + 1, :])
    xpx[0, :, :] = jnp.zeros_like(xpx[0, :, :])
    xpx[_H + 1, :, :] = jnp.zeros_like(xpx[_H + 1, :, :])
    xps[:, 0, :] = jnp.zeros_like(xps[:, 0, :])
    xps[:, _H + 1, :] = jnp.zeros_like(xps[:, _H + 1, :])
    xps[0, :, :] = jnp.zeros_like(xps[0, :, :])
    xps[_H + 1, :, :] = jnp.zeros_like(xps[_H + 1, :, :])
    for img in range(_B1):
        xu = _upsample2x(x_ref[img].astype(jnp.float32))
        xpx[1:_H + 1, 1:_H + 1, :] = xu.astype(jnp.bfloat16)
        xps[1:_H + 1, 1:_H + 1, :] = sk_ref[img]
        for t in range(9):
            dy, dx = divmod(t, 3)
            col[:, :, t * _IC:(t + 1) * _IC] = xpx[dy:dy + _H, dx:dx + _H, :]
            col[:, :, 9 * _IC + t * _OC:9 * _IC + (t + 1) * _OC] = (
                xps[dy:dy + _H, dx:dx + _H, :])
        acc = jnp.dot(col[...].reshape(_M, _K1), w_ref[...],
                      preferred_element_type=jnp.float32)
        _relu_stats_store(acc, bmap_ref, y_ref.at[img], st_ref.at[img])


def _conv2_body(y1_ref, w_ref, bmap_ref, y_ref, st_ref, xpy, col):
    xpy[:, 0, :] = jnp.zeros_like(xpy[:, 0, :])
    xpy[:, _H + 1, :] = jnp.zeros_like(xpy[:, _H + 1, :])
    xpy[0, :, :] = jnp.zeros_like(xpy[0, :, :])
    xpy[_H + 1, :, :] = jnp.zeros_like(xpy[_H + 1, :, :])
    for img in range(_B):
        xpy[1:_H + 1, 1:_H + 1, :] = y1_ref[img]
        for t in range(9):
            dy, dx = divmod(t, 3)
            col[:, :, t * _OC:(t + 1) * _OC] = xpy[dy:dy + _H, dx:dx + _H, :]
        acc = jnp.dot(col[...].reshape(_M, _K2), w_ref[...],
                      preferred_element_type=jnp.float32)
        _relu_stats_store(acc, bmap_ref, y_ref.at[img], st_ref.at[img])


def _conv1_call(x, skip, w, bmap):
    n = x.shape[0]
    return pl.pallas_call(
        _conv1_body,
        out_shape=(jax.ShapeDtypeStruct((n, _M, _OC), jnp.bfloat16),
                   jax.ShapeDtypeStruct((n, 2, _OC), jnp.float32)),
        grid=(n // _B1,),
        in_specs=[
            pl.BlockSpec((_B1, _H // 2, _H // 2, _IC), lambda i: (i, 0, 0, 0)),
            pl.BlockSpec((_B1, _H, _H, _OC), lambda i: (i, 0, 0, 0)),
            pl.BlockSpec((_K1, _OC), lambda i: (0, 0)),
            pl.BlockSpec((_M, _OC), lambda i: (0, 0)),
        ],
        out_specs=(
            pl.BlockSpec((_B1, _M, _OC), lambda i: (i, 0, 0)),
            pl.BlockSpec((_B1, 2, _OC), lambda i: (i, 0, 0)),
        ),
        scratch_shapes=[
            pltpu.VMEM((_H + 2, _H + 2, _IC), jnp.bfloat16),
            pltpu.VMEM((_H + 2, _H + 2, _OC), jnp.bfloat16),
            pltpu.VMEM((_H, _H, _K1), jnp.bfloat16),
        ],
        compiler_params=pltpu.CompilerParams(
            dimension_semantics=("parallel",)),
    )(x, skip, w, bmap)


def _conv2_call(y1, w, bmap):
    n = y1.shape[0]
    return pl.pallas_call(
        _conv2_body,
        out_shape=(jax.ShapeDtypeStruct((n, _M, _OC), jnp.bfloat16),
                   jax.ShapeDtypeStruct((n, 2, _OC), jnp.float32)),
        grid=(n // _B,),
        in_specs=[
            pl.BlockSpec((_B, _H, _H, _OC), lambda i: (i, 0, 0, 0)),
            pl.BlockSpec((_K2, _OC), lambda i: (0, 0)),
            pl.BlockSpec((_M, _OC), lambda i: (0, 0)),
        ],
        out_specs=(
            pl.BlockSpec((_B, _M, _OC), lambda i: (i, 0, 0)),
            pl.BlockSpec((_B, 2, _OC), lambda i: (i, 0, 0)),
        ),
        scratch_shapes=[
            pltpu.VMEM((_H + 2, _H + 2, _OC), jnp.bfloat16),
            pltpu.VMEM((_H, _H, _K2), jnp.bfloat16),
        ],
        compiler_params=pltpu.CompilerParams(
            dimension_semantics=("parallel",)),
    )(y1, w, bmap)


def _epi_body(st_ref, g2_ref, b2_ref, y2_ref, o_ref, count):
    st = jnp.sum(st_ref[...], axis=0)
    s2, t2 = _bn_rows(st, g2_ref[...], b2_ref[...], count)
    for img in range(_B):
        z = y2_ref[img].astype(jnp.float32) * s2 + t2
        o_ref[img] = jnp.transpose(z.reshape(_H, _H, _OC), (2, 0, 1))


def _epi_call(st_all, g2, b2, y2, count):
    n = y2.shape[0]
    return pl.pallas_call(
        functools.partial(_epi_body, count=count),
        out_shape=jax.ShapeDtypeStruct((n, _OC, _H, _H), jnp.float32),
        grid=(n // _B,),
        in_specs=[
            pl.BlockSpec((n, 2, _OC), lambda i: (0, 0, 0)),
            pl.BlockSpec((1, _OC), lambda i: (0, 0)),
            pl.BlockSpec((1, _OC), lambda i: (0, 0)),
            pl.BlockSpec((_B, _M, _OC), lambda i: (i, 0, 0)),
        ],
        out_specs=pl.BlockSpec((_B, _OC, _H, _H), lambda i: (i, 0, 0, 0)),
        compiler_params=pltpu.CompilerParams(
            dimension_semantics=("parallel",)),
    )(st_all, g2, b2, y2)


def kernel(up_w, up_b, c1_w, c1_b, c2_w, c2_b,
           bn1_g, bn1_b, bn2_g, bn2_b, x, skip):
    n = x.shape[0]
    count = float(n * _M)

    x_nhwc = jnp.transpose(x, (0, 2, 3, 1)).astype(jnp.bfloat16)
    sk_nhwc = jnp.transpose(skip, (0, 2, 3, 1)).astype(jnp.bfloat16)
    w1, bmap1 = _prep1_call(up_w, up_b[None, :], c1_w, c1_b[None, :])
    y1, st1 = _conv1_call(x_nhwc, sk_nhwc, w1, bmap1)

    w2, bmap2 = _prep2_call(st1, bn1_g[None, :], bn1_b[None, :],
                            c2_w, c2_b[None, :], count)
    y2, st2 = _conv2_call(y1.reshape(n, _H, _H, _OC), w2, bmap2)

    return _epi_call(st2, bn2_g[None, :], bn2_b[None, :], y2, count)
